# baseline (speedup 1.0000x reference)
import jax
import jax.numpy as jnp
from jax.experimental import pallas as pl
from jax.experimental.pallas import tpu as pltpu

LORA_ALPHA = 128.0


def _body(x_ref, g_ref, wd_ref, wu_ref, o_ref, *, scale):
    xb = x_ref[...]
    gs = jnp.sum(xb * g_ref[...], axis=-1, keepdims=True)
    h = jnp.dot(xb, wd_ref[...], preferred_element_type=jnp.float32)
    h = h * (jax.nn.sigmoid(gs) * scale)
    o_ref[...] = jnp.dot(h.astype(jnp.bfloat16), wu_ref[...],
                         preferred_element_type=jnp.float32)


def kernel(x, W_down, W_up, input_gate):
    B, S, D_IN = x.shape
    R = W_down.shape[0]
    D_OUT = W_up.shape[0]
    scale = LORA_ALPHA / R

    T = B * S
    TILE = 1024
    xf = x.reshape(T, D_IN)
    wd = W_down.T
    wu = W_up.T.astype(jnp.bfloat16)
    g = input_gate.reshape(1, D_IN)

    out = pl.pallas_call(
        lambda *refs: _body(*refs, scale=scale),
        grid=(T // TILE,),
        in_specs=[
            pl.BlockSpec((TILE, D_IN), lambda i: (i, 0)),
            pl.BlockSpec((1, D_IN), lambda i: (0, 0)),
            pl.BlockSpec((D_IN, R), lambda i: (0, 0)),
            pl.BlockSpec((R, D_OUT), lambda i: (0, 0)),
        ],
        out_specs=pl.BlockSpec((TILE, D_OUT), lambda i: (i, 0)),
        out_shape=jax.ShapeDtypeStruct((T, D_OUT), jnp.float32),
        compiler_params=pltpu.CompilerParams(
            dimension_semantics=("parallel",),
        ),
    )(xf, g, wd, wu)

    return out.reshape(B, S, D_OUT)


# speedup vs baseline: 1.0937x; 1.0937x over previous
import functools

import jax
import jax.numpy as jnp
from jax.experimental import pallas as pl
from jax.experimental.pallas import tpu as pltpu

LORA_ALPHA = 128.0
N_IN = 3
N_OUT = 2


def _body(xf_ref, g_ref, wd_ref, wu_ref, o_ref, xbuf, obuf, insem, outsem,
          *, scale, tile, nstep):
    def in_copy(step, slot):
        return pltpu.make_async_copy(
            xf_ref.at[pl.ds(step * tile, tile), :], xbuf.at[slot],
            insem.at[slot])

    def out_copy(step, slot):
        return pltpu.make_async_copy(
            obuf.at[slot], o_ref.at[pl.ds(step * tile, tile), :],
            outsem.at[slot])

    for s in range(N_IN):
        in_copy(s, s).start()

    def step_fn(i, carry):
        islot = jax.lax.rem(i, N_IN)
        oslot = jax.lax.rem(i, N_OUT)
        in_copy(i, islot).wait()

        @pl.when(i >= N_OUT)
        def _():
            out_copy(i - N_OUT, oslot).wait()

        xb = xbuf[islot]
        gs = jnp.sum(xb * g_ref[...], axis=-1, keepdims=True)
        h = jnp.dot(xb, wd_ref[...], preferred_element_type=jnp.float32)
        h = h * (jax.nn.sigmoid(gs) * scale)
        obuf[oslot] = jnp.dot(h, wu_ref[...],
                              preferred_element_type=jnp.float32)
        out_copy(i, oslot).start()

        @pl.when(i + N_IN < nstep)
        def _():
            in_copy(i + N_IN, islot).start()

        return carry

    jax.lax.fori_loop(0, nstep, step_fn, 0)

    for k in range(N_OUT):
        step = nstep - N_OUT + k
        out_copy(step, step % N_OUT).wait()


def kernel(x, W_down, W_up, input_gate):
    B, S, D_IN = x.shape
    R = W_down.shape[0]
    D_OUT = W_up.shape[0]
    scale = LORA_ALPHA / R

    T = B * S
    TILE = 1024
    nstep = T // TILE
    xf = x.reshape(T, D_IN)
    wd = W_down.T
    wu = W_up.T
    g = input_gate.reshape(1, D_IN)

    out = pl.pallas_call(
        functools.partial(_body, scale=scale, tile=TILE, nstep=nstep),
        in_specs=[
            pl.BlockSpec(memory_space=pltpu.MemorySpace.HBM),
            pl.BlockSpec(memory_space=pltpu.MemorySpace.VMEM),
            pl.BlockSpec(memory_space=pltpu.MemorySpace.VMEM),
            pl.BlockSpec(memory_space=pltpu.MemorySpace.VMEM),
        ],
        out_specs=pl.BlockSpec(memory_space=pltpu.MemorySpace.HBM),
        out_shape=jax.ShapeDtypeStruct((T, D_OUT), jnp.float32),
        scratch_shapes=[
            pltpu.MemorySpace.VMEM((N_IN, TILE, D_IN), jnp.float32),
            pltpu.MemorySpace.VMEM((N_OUT, TILE, D_OUT), jnp.float32),
            pltpu.SemaphoreType.DMA((N_IN,)),
            pltpu.SemaphoreType.DMA((N_OUT,)),
        ],
    )(xf, g, wd, wu)

    return out.reshape(B, S, D_OUT)


# speedup vs baseline: 1.1427x; 1.0448x over previous
import functools

import jax
import jax.numpy as jnp
from jax.experimental import pallas as pl
from jax.experimental.pallas import tpu as pltpu

LORA_ALPHA = 128.0
N_IN = 3
N_OUT = 3


def _body(xf_ref, g_ref, wd_ref, wu_ref, o_ref, xbuf, obuf, insem, outsem,
          *, scale, tile, nstep):
    def in_copy(step, slot):
        return pltpu.make_async_copy(
            xf_ref.at[pl.ds(step * tile, tile), :], xbuf.at[slot],
            insem.at[slot])

    def out_copy(step, slot):
        return pltpu.make_async_copy(
            obuf.at[slot], o_ref.at[pl.ds(step * tile, tile), :],
            outsem.at[slot])

    for s in range(N_IN):
        in_copy(s, s).start()

    def step_fn(i, carry):
        islot = jax.lax.rem(i, N_IN)
        oslot = jax.lax.rem(i, N_OUT)
        in_copy(i, islot).wait()

        @pl.when(i >= N_OUT)
        def _():
            out_copy(i - N_OUT, oslot).wait()

        xb = xbuf[islot]
        gs = jnp.sum(xb * g_ref[...], axis=-1, keepdims=True)
        h = jnp.dot(xb, wd_ref[...], preferred_element_type=jnp.float32)
        h = h * (jax.nn.sigmoid(gs) * scale)
        obuf[oslot] = jnp.dot(h, wu_ref[...],
                              preferred_element_type=jnp.float32)
        out_copy(i, oslot).start()

        @pl.when(i + N_IN < nstep)
        def _():
            in_copy(i + N_IN, islot).start()

        return carry

    jax.lax.fori_loop(0, nstep, step_fn, 0)

    for k in range(N_OUT):
        step = nstep - N_OUT + k
        out_copy(step, step % N_OUT).wait()


def kernel(x, W_down, W_up, input_gate):
    B, S, D_IN = x.shape
    R = W_down.shape[0]
    D_OUT = W_up.shape[0]
    scale = LORA_ALPHA / R

    T = B * S
    TILE = 1024
    nstep = T // TILE
    xf = x.reshape(T, D_IN)
    wd = W_down.T
    wu = W_up.T
    g = input_gate.reshape(1, D_IN)

    out = pl.pallas_call(
        functools.partial(_body, scale=scale, tile=TILE, nstep=nstep),
        in_specs=[
            pl.BlockSpec(memory_space=pltpu.MemorySpace.HBM),
            pl.BlockSpec(memory_space=pltpu.MemorySpace.VMEM),
            pl.BlockSpec(memory_space=pltpu.MemorySpace.VMEM),
            pl.BlockSpec(memory_space=pltpu.MemorySpace.VMEM),
        ],
        out_specs=pl.BlockSpec(memory_space=pltpu.MemorySpace.HBM),
        out_shape=jax.ShapeDtypeStruct((T, D_OUT), jnp.float32),
        scratch_shapes=[
            pltpu.MemorySpace.VMEM((N_IN, TILE, D_IN), jnp.float32),
            pltpu.MemorySpace.VMEM((N_OUT, TILE, D_OUT), jnp.float32),
            pltpu.SemaphoreType.DMA((N_IN,)),
            pltpu.SemaphoreType.DMA((N_OUT,)),
        ],
    )(xf, g, wd, wu)

    return out.reshape(B, S, D_OUT)


# speedup vs baseline: 1.1487x; 1.0052x over previous
import functools

import jax
import jax.numpy as jnp
from jax.experimental import pallas as pl
from jax.experimental.pallas import tpu as pltpu

LORA_ALPHA = 128.0
N_IN = 6
N_OUT = 6


def _body(xf_ref, g_ref, wd_ref, wu_ref, o_ref, xbuf, obuf, insem, outsem,
          *, scale, tile, nstep):
    def in_copy(step, slot):
        return pltpu.make_async_copy(
            xf_ref.at[pl.ds(step * tile, tile), :], xbuf.at[slot],
            insem.at[slot])

    def out_copy(step, slot):
        return pltpu.make_async_copy(
            obuf.at[slot], o_ref.at[pl.ds(step * tile, tile), :],
            outsem.at[slot])

    for s in range(N_IN):
        in_copy(s, s).start()

    def step_fn(i, carry):
        islot = jax.lax.rem(i, N_IN)
        oslot = jax.lax.rem(i, N_OUT)
        in_copy(i, islot).wait()

        @pl.when(i >= N_OUT)
        def _():
            out_copy(i - N_OUT, oslot).wait()

        xb = xbuf[islot]
        gs = jnp.sum(xb * g_ref[...], axis=-1, keepdims=True)
        h = jnp.dot(xb, wd_ref[...], preferred_element_type=jnp.float32)
        h = h * (jax.nn.sigmoid(gs) * scale)
        obuf[oslot] = jnp.dot(h, wu_ref[...],
                              preferred_element_type=jnp.float32)
        out_copy(i, oslot).start()

        @pl.when(i + N_IN < nstep)
        def _():
            in_copy(i + N_IN, islot).start()

        return carry

    jax.lax.fori_loop(0, nstep, step_fn, 0)

    for k in range(N_OUT):
        step = nstep - N_OUT + k
        out_copy(step, step % N_OUT).wait()


def kernel(x, W_down, W_up, input_gate):
    B, S, D_IN = x.shape
    R = W_down.shape[0]
    D_OUT = W_up.shape[0]
    scale = LORA_ALPHA / R

    T = B * S
    TILE = 512
    nstep = T // TILE
    xf = x.reshape(T, D_IN)
    wd = W_down.T
    wu = W_up.T
    g = input_gate.reshape(1, D_IN)

    out = pl.pallas_call(
        functools.partial(_body, scale=scale, tile=TILE, nstep=nstep),
        in_specs=[
            pl.BlockSpec(memory_space=pltpu.MemorySpace.HBM),
            pl.BlockSpec(memory_space=pltpu.MemorySpace.VMEM),
            pl.BlockSpec(memory_space=pltpu.MemorySpace.VMEM),
            pl.BlockSpec(memory_space=pltpu.MemorySpace.VMEM),
        ],
        out_specs=pl.BlockSpec(memory_space=pltpu.MemorySpace.HBM),
        out_shape=jax.ShapeDtypeStruct((T, D_OUT), jnp.float32),
        scratch_shapes=[
            pltpu.MemorySpace.VMEM((N_IN, TILE, D_IN), jnp.float32),
            pltpu.MemorySpace.VMEM((N_OUT, TILE, D_OUT), jnp.float32),
            pltpu.SemaphoreType.DMA((N_IN,)),
            pltpu.SemaphoreType.DMA((N_OUT,)),
        ],
    )(xf, g, wd, wu)

    return out.reshape(B, S, D_OUT)


# speedup vs baseline: 1.1496x; 1.0007x over previous
import functools

import jax
import jax.numpy as jnp
from jax.experimental import pallas as pl
from jax.experimental.pallas import tpu as pltpu

LORA_ALPHA = 128.0
N_IN = 8
N_OUT = 5


def _body(xf_ref, g_ref, wd_ref, wu_ref, o_ref, xbuf, obuf, insem, outsem,
          *, scale, tile, nstep):
    def in_copy(step, slot):
        return pltpu.make_async_copy(
            xf_ref.at[pl.ds(step * tile, tile), :], xbuf.at[slot],
            insem.at[slot])

    def out_copy(step, slot):
        return pltpu.make_async_copy(
            obuf.at[slot], o_ref.at[pl.ds(step * tile, tile), :],
            outsem.at[slot])

    for s in range(N_IN):
        in_copy(s, s).start()

    def step_fn(i, carry):
        islot = jax.lax.rem(i, N_IN)
        oslot = jax.lax.rem(i, N_OUT)
        in_copy(i, islot).wait()

        @pl.when(i >= N_OUT)
        def _():
            out_copy(i - N_OUT, oslot).wait()

        xb = xbuf[islot]
        gs = jnp.sum(xb * g_ref[...], axis=-1, keepdims=True)
        h = jnp.dot(xb, wd_ref[...], preferred_element_type=jnp.float32)
        h = h * (jax.nn.sigmoid(gs) * scale)
        obuf[oslot] = jnp.dot(h, wu_ref[...],
                              preferred_element_type=jnp.float32)
        out_copy(i, oslot).start()

        @pl.when(i + N_IN < nstep)
        def _():
            in_copy(i + N_IN, islot).start()

        return carry

    jax.lax.fori_loop(0, nstep, step_fn, 0)

    for k in range(N_OUT):
        step = nstep - N_OUT + k
        out_copy(step, step % N_OUT).wait()


def kernel(x, W_down, W_up, input_gate):
    B, S, D_IN = x.shape
    R = W_down.shape[0]
    D_OUT = W_up.shape[0]
    scale = LORA_ALPHA / R

    T = B * S
    TILE = 512
    nstep = T // TILE
    xf = x.reshape(T, D_IN)
    wd = W_down.T
    wu = W_up.T
    g = input_gate.reshape(1, D_IN)

    out = pl.pallas_call(
        functools.partial(_body, scale=scale, tile=TILE, nstep=nstep),
        in_specs=[
            pl.BlockSpec(memory_space=pltpu.MemorySpace.HBM),
            pl.BlockSpec(memory_space=pltpu.MemorySpace.VMEM),
            pl.BlockSpec(memory_space=pltpu.MemorySpace.VMEM),
            pl.BlockSpec(memory_space=pltpu.MemorySpace.VMEM),
        ],
        out_specs=pl.BlockSpec(memory_space=pltpu.MemorySpace.HBM),
        out_shape=jax.ShapeDtypeStruct((T, D_OUT), jnp.float32),
        scratch_shapes=[
            pltpu.MemorySpace.VMEM((N_IN, TILE, D_IN), jnp.float32),
            pltpu.MemorySpace.VMEM((N_OUT, TILE, D_OUT), jnp.float32),
            pltpu.SemaphoreType.DMA((N_IN,)),
            pltpu.SemaphoreType.DMA((N_OUT,)),
        ],
    )(xf, g, wd, wu)

    return out.reshape(B, S, D_OUT)
